# Initial kernel scaffold; baseline (speedup 1.0000x reference)
import jax
import jax.numpy as jnp
from jax.experimental import pallas as pl


def kernel(tokens, table):
    raise NotImplementedError("write your pallas kernel here")


# speedup vs baseline: 3.1936x; 3.1936x over previous
import functools

import jax
import jax.numpy as jnp
from jax import lax
from jax.experimental import pallas as pl
from jax.experimental.pallas import tpu as pltpu
from jax.experimental.pallas import tpu_sc as plsc

D = 64
SCALE = 8.0
NC = 2
NS = 16
NW = NC * NS
L = 16
CHUNK = 800


@functools.partial(jax.jit, static_argnums=())
def _sc_embed(idx_flat, table):
    B = idx_flat.shape[0]
    b_per_w = B // NW
    n_chunks = b_per_w // CHUNK
    mesh = plsc.VectorSubcoreMesh(core_axis_name="c", subcore_axis_name="s")

    @functools.partial(
        pl.kernel,
        mesh=mesh,
        out_type=jax.ShapeDtypeStruct((B, D), jnp.float32),
        scratch_types=[
            pltpu.VMEM((CHUNK,), jnp.int32),
            pltpu.VMEM((CHUNK, D), jnp.float32),
            pltpu.SemaphoreType.DMA,
        ],
        compiler_params=pltpu.CompilerParams(use_tc_tiling_on_sc=False),
    )
    def k(table_hbm, idx_hbm, out_hbm, idx_v, rows_v, sem):
        wid = lax.axis_index("s") * NC + lax.axis_index("c")
        base = wid * b_per_w

        def chunk_body(j, carry):
            off = base + j * CHUNK
            pltpu.sync_copy(idx_hbm.at[pl.ds(off, CHUNK)], idx_v)
            pltpu.async_copy(table_hbm.at[idx_v], rows_v, sem).wait()

            def scale_body(r, c):
                for q in range(D // L):
                    rows_v[r, pl.ds(q * L, L)] = rows_v[r, pl.ds(q * L, L)] * SCALE
                return c

            lax.fori_loop(0, CHUNK, scale_body, 0)
            pltpu.sync_copy(rows_v, out_hbm.at[pl.ds(off, CHUNK)])
            return carry

        lax.fori_loop(0, n_chunks, chunk_body, 0)

    return k(table, idx_flat)


def kernel(tokens, table):
    idx = tokens.reshape(-1)
    out = _sc_embed(idx, table)
    return out.reshape(tokens.shape[0], tokens.shape[1], D)


# speedup vs baseline: 3.5986x; 1.1268x over previous
import functools

import jax
import jax.numpy as jnp
from jax import lax
from jax.experimental import pallas as pl
from jax.experimental.pallas import tpu as pltpu
from jax.experimental.pallas import tpu_sc as plsc

D = 64
SCALE = 8.0
NC = 2
NS = 16
NW = NC * NS
L = 16
CHUNK = 800


def _sc_embed(idx_flat, table):
    B = idx_flat.shape[0]
    b_per_w = B // NW
    n_chunks = b_per_w // CHUNK
    mesh = plsc.VectorSubcoreMesh(core_axis_name="c", subcore_axis_name="s")

    @functools.partial(
        pl.kernel,
        mesh=mesh,
        out_type=jax.ShapeDtypeStruct((B, D), jnp.float32),
        scratch_types=[
            pltpu.VMEM((b_per_w,), jnp.int32),
            pltpu.VMEM((CHUNK, D), jnp.float32),
            pltpu.VMEM((CHUNK, D), jnp.float32),
            pltpu.SemaphoreType.DMA,
            pltpu.SemaphoreType.DMA,
            pltpu.SemaphoreType.DMA,
            pltpu.SemaphoreType.DMA,
        ],
        compiler_params=pltpu.CompilerParams(use_tc_tiling_on_sc=False),
    )
    def k(table_hbm, idx_hbm, out_hbm, idx_all, rows0, rows1, g0, g1, w0, w1):
        wid = lax.axis_index("s") * NC + lax.axis_index("c")
        base = wid * b_per_w
        rows = (rows0, rows1)
        gsem = (g0, g1)
        wsem = (w0, w1)

        pltpu.sync_copy(idx_hbm.at[pl.ds(base, b_per_w)], idx_all)

        def gather(j, buf):
            return pltpu.async_copy(
                table_hbm.at[idx_all.at[pl.ds(j * CHUNK, CHUNK)]],
                rows[buf], gsem[buf])

        def writeback(j, buf):
            return pltpu.async_copy(
                rows[buf], out_hbm.at[pl.ds(base + j * CHUNK, CHUNK)],
                wsem[buf])

        g_handles = [None, None]
        w_handles = [None, None]
        g_handles[0] = gather(0, 0)
        for j in range(n_chunks):
            cur = j & 1
            nxt = cur ^ 1
            if j + 1 < n_chunks:
                if w_handles[nxt] is not None:
                    w_handles[nxt].wait()
                g_handles[nxt] = gather(j + 1, nxt)
            g_handles[cur].wait()

            r = rows[cur]

            @plsc.parallel_loop(0, CHUNK, 1, unroll=8)
            def _scale(row):
                for q in range(D // L):
                    r[row, pl.ds(q * L, L)] = r[row, pl.ds(q * L, L)] * SCALE

            w_handles[cur] = writeback(j, cur)
        w_handles[0].wait()
        w_handles[1].wait()

    return k(table, idx_flat)


def kernel(tokens, table):
    idx = tokens.reshape(-1)
    out = _sc_embed(idx, table)
    return out.reshape(tokens.shape[0], tokens.shape[1], D)
